# baseline (speedup 1.0000x reference)
import jax
import jax.numpy as jnp
from jax.experimental import pallas as pl
from jax.experimental.pallas import tpu as pltpu
from jax.experimental.pallas import tpu_sc as plsc

VOCAB = 100000
D = 64
CTX = 20
B = 1024
TN = 1024
NT = (VOCAB + TN - 1) // TN
VP = NT * TN
GW = 128
NIDX = CTX * B


def _sc_gather(emb2, idx2):
    @pl.kernel(
        out_type=jax.ShapeDtypeStruct((NIDX, 2 * D), emb2.dtype),
        mesh=plsc.VectorSubcoreMesh(core_axis_name="core",
                                    subcore_axis_name="subcore"),
    )
    def gather_kernel(emb_hbm, i_hbm, o_hbm):
        def body(i_vmem, o_vmem):
            pltpu.sync_copy(emb_hbm.at[i_vmem.at[0]], o_vmem)

        pltpu.emit_pipeline(
            body,
            grid=(NIDX // GW,),
            in_specs=[pl.BlockSpec((1, GW), index_map=lambda i: (0, i))],
            out_specs=[pl.BlockSpec((GW, 2 * D), index_map=lambda i: (i, 0))],
            core_axis_name=("core", "subcore"),
            dimension_semantics=(pltpu.PARALLEL,),
        )(i_hbm, o_hbm)

    return gather_kernel(emb2, idx2)


def _lse_kernel(g_ref, par_ref, wt_ref, b_ref, lse_ref, x_ref,
                m_scr, s_scr, x_scr):
    i = pl.program_id(0)

    @pl.when(i == 0)
    def _init():
        x = jnp.zeros((B, D), jnp.float32)
        for k in range(CTX):
            blk = g_ref[k * B:(k + 1) * B, :]
            par = par_ref[k * B:(k + 1) * B, :]
            x = x + jnp.where(par != 0, blk[:, D:2 * D], blk[:, 0:D])
        x_scr[...] = x
        m_scr[...] = jnp.full((B, 1), -jnp.inf, jnp.float32)
        s_scr[...] = jnp.zeros((B, 1), jnp.float32)

    xb = x_scr[...].astype(jnp.bfloat16)
    logits = jax.lax.dot_general(
        xb, wt_ref[...], (((1,), (0,)), ((), ())),
        preferred_element_type=jnp.float32) + b_ref[...]
    tile_max = jnp.max(logits, axis=1, keepdims=True)
    m_old = m_scr[...]
    m_new = jnp.maximum(m_old, tile_max)
    s_scr[...] = s_scr[...] * jnp.exp(m_old - m_new) + jnp.sum(
        jnp.exp(logits - m_new), axis=1, keepdims=True)
    m_scr[...] = m_new

    @pl.when(i == NT - 1)
    def _fin():
        lse_ref[...] = m_scr[...] + jnp.log(s_scr[...])
        x_ref[...] = x_scr[...]


def _out_kernel(x_ref, wt_ref, b_ref, lse_ref, o_ref):
    xb = x_ref[...].astype(jnp.bfloat16)
    logits = jax.lax.dot_general(
        xb, wt_ref[...], (((1,), (0,)), ((), ())),
        preferred_element_type=jnp.float32) + b_ref[...]
    o_ref[...] = logits - lse_ref[...]


def kernel(inputs, embedding, W, b):
    idx = inputs.reshape(NIDX).astype(jnp.int32)
    emb2 = embedding.reshape(VOCAB // 2, 2 * D)
    g = _sc_gather(emb2, (idx // 2).reshape(1, NIDX))
    par = (idx % 2).reshape(NIDX, 1)
    wt = jnp.pad(W.T.astype(jnp.bfloat16), ((0, 0), (0, VP - VOCAB)))
    bp = jnp.pad(b, (0, VP - VOCAB), constant_values=-1e30).reshape(1, VP)

    lse, x = pl.pallas_call(
        _lse_kernel,
        grid=(NT,),
        in_specs=[
            pl.BlockSpec((NIDX, 2 * D), lambda i: (0, 0)),
            pl.BlockSpec((NIDX, 1), lambda i: (0, 0)),
            pl.BlockSpec((D, TN), lambda i: (0, i)),
            pl.BlockSpec((1, TN), lambda i: (0, i)),
        ],
        out_specs=[
            pl.BlockSpec((B, 1), lambda i: (0, 0)),
            pl.BlockSpec((B, D), lambda i: (0, 0)),
        ],
        out_shape=[
            jax.ShapeDtypeStruct((B, 1), jnp.float32),
            jax.ShapeDtypeStruct((B, D), jnp.float32),
        ],
        scratch_shapes=[
            pltpu.VMEM((B, 1), jnp.float32),
            pltpu.VMEM((B, 1), jnp.float32),
            pltpu.VMEM((B, D), jnp.float32),
        ],
        compiler_params=pltpu.CompilerParams(
            dimension_semantics=("arbitrary",)),
    )(g, par, wt, bp)

    out = pl.pallas_call(
        _out_kernel,
        grid=(NT,),
        in_specs=[
            pl.BlockSpec((B, D), lambda i: (0, 0)),
            pl.BlockSpec((D, TN), lambda i: (0, i)),
            pl.BlockSpec((1, TN), lambda i: (0, i)),
            pl.BlockSpec((B, 1), lambda i: (0, 0)),
        ],
        out_specs=pl.BlockSpec((B, TN), lambda i: (0, i)),
        out_shape=jax.ShapeDtypeStruct((B, VOCAB), jnp.float32),
        compiler_params=pltpu.CompilerParams(
            dimension_semantics=("arbitrary",)),
    )(x, wt, bp, lse)
    return out
